# Initial kernel scaffold; baseline (speedup 1.0000x reference)
import jax
import jax.numpy as jnp
from jax.experimental import pallas as pl


def kernel(a, b, emb, W1, b1, W2, b2, W3, b3):
    raise NotImplementedError("write your pallas kernel here")


# speedup vs baseline: 8.7749x; 8.7749x over previous
import functools

import jax
import jax.numpy as jnp
from jax import lax
from jax.experimental import pallas as pl
from jax.experimental.pallas import tpu as pltpu
from jax.experimental.pallas import tpu_sc as plsc

EMBD = 128
SEQ = 50
LANES = 16
NCORES = 2
NSUB = 16
NW = NCORES * NSUB


def _sc_pool(emb, idx_flat, n_samples):
    per_w = n_samples // NW
    CH = 8
    n_chunks = per_w // CH
    mesh = plsc.VectorSubcoreMesh(core_axis_name="c", subcore_axis_name="s")

    @functools.partial(
        pl.kernel,
        mesh=mesh,
        out_type=jax.ShapeDtypeStruct((n_samples, EMBD), jnp.float32),
        scratch_types=[
            pltpu.VMEM((CH * SEQ,), jnp.int32),
            pltpu.VMEM((CH * SEQ, EMBD), jnp.float32),
            pltpu.VMEM((CH, EMBD), jnp.float32),
            pltpu.SemaphoreType.DMA,
        ],
    )
    def pool_kernel(emb_hbm, idx_hbm, out_hbm, idx_v, rows_v, acc_v, sem):
        wid = lax.axis_index("s") * NCORES + lax.axis_index("c")
        base = wid * per_w

        def chunk_body(ci, carry):
            s0 = base + ci * CH
            pltpu.sync_copy(idx_hbm.at[pl.ds(s0 * SEQ, CH * SEQ)], idx_v)
            pltpu.async_copy(emb_hbm.at[idx_v], rows_v, sem).wait()
            for s in range(CH):
                accs = tuple(
                    rows_v[s * SEQ, pl.ds(d * LANES, LANES)]
                    for d in range(EMBD // LANES)
                )

                def jbody(j, accs):
                    return tuple(
                        accs[d] + rows_v[s * SEQ + j, pl.ds(d * LANES, LANES)]
                        for d in range(EMBD // LANES)
                    )

                accs = lax.fori_loop(1, SEQ, jbody, accs)
                for d in range(EMBD // LANES):
                    acc_v[s, pl.ds(d * LANES, LANES)] = accs[d]
            pltpu.sync_copy(acc_v, out_hbm.at[pl.ds(s0, CH)])
            return carry

        lax.fori_loop(0, n_chunks, chunk_body, 0)

    return pool_kernel(emb, idx_flat)


def _tc_mlp_cos(pooled, W1, b1, W2, b2, W3, b3, batch):
    TB = 512
    grid = batch // TB

    def body(ea_ref, eb_ref, w1, bb1, w2, bb2, w3, bb3, out_ref):
        def mlp(x):
            h = jnp.tanh(
                lax.dot_general(x, w1[...], (((1,), (1,)), ((), ())),
                                preferred_element_type=jnp.float32) + bb1[...])
            h = jnp.tanh(
                lax.dot_general(h, w2[...], (((1,), (1,)), ((), ())),
                                preferred_element_type=jnp.float32) + bb2[...])
            h = jnp.tanh(
                lax.dot_general(h, w3[...], (((1,), (1,)), ((), ())),
                                preferred_element_type=jnp.float32) + bb3[...])
            return h

        fa = mlp(ea_ref[...])
        fb = mlp(eb_ref[...])
        eps = 1e-8
        na = jnp.maximum(jnp.sqrt(jnp.sum(fa * fa, axis=1, keepdims=True)), eps)
        nb = jnp.maximum(jnp.sqrt(jnp.sum(fb * fb, axis=1, keepdims=True)), eps)
        dot = jnp.sum(fa * fb, axis=1, keepdims=True)
        out_ref[...] = dot / (na * nb)

    out = pl.pallas_call(
        body,
        grid=(grid,),
        in_specs=[
            pl.BlockSpec((TB, EMBD), lambda i: (i, 0)),
            pl.BlockSpec((TB, EMBD), lambda i: (i + grid, 0)),
            pl.BlockSpec(W1.shape, lambda i: (0, 0)),
            pl.BlockSpec((1, 256), lambda i: (0, 0)),
            pl.BlockSpec(W2.shape, lambda i: (0, 0)),
            pl.BlockSpec((1, 128), lambda i: (0, 0)),
            pl.BlockSpec(W3.shape, lambda i: (0, 0)),
            pl.BlockSpec((1, 64), lambda i: (0, 0)),
        ],
        out_specs=pl.BlockSpec((TB, 1), lambda i: (i, 0)),
        out_shape=jax.ShapeDtypeStruct((batch, 1), jnp.float32),
    )(pooled, pooled, W1, b1.reshape(1, -1), W2, b2.reshape(1, -1),
      W3, b3.reshape(1, -1))
    return out.reshape(-1)


def kernel(a, b, emb, W1, b1, W2, b2, W3, b3):
    batch = a.shape[0]
    idx = jnp.concatenate([a, b], axis=0).astype(jnp.int32).reshape(-1)
    pooled = _sc_pool(emb, idx, 2 * batch)
    return _tc_mlp_cos(pooled, W1, b1, W2, b2, W3, b3, batch)


# speedup vs baseline: 13.4156x; 1.5289x over previous
import functools

import jax
import jax.numpy as jnp
from jax import lax
from jax.experimental import pallas as pl
from jax.experimental.pallas import tpu as pltpu
from jax.experimental.pallas import tpu_sc as plsc

EMBD = 128
SEQ = 50
LANES = 16
NCORES = 2
NSUB = 16
NW = NCORES * NSUB


def _sc_pool(emb, idx_flat, n_samples):
    per_w = n_samples // NW
    CH = 8
    PAIRS = per_w // (2 * CH)
    D = EMBD // LANES
    mesh = plsc.VectorSubcoreMesh(core_axis_name="c", subcore_axis_name="s")

    @functools.partial(
        pl.kernel,
        mesh=mesh,
        out_type=jax.ShapeDtypeStruct((n_samples, EMBD), jnp.float32),
        scratch_types=[
            pltpu.VMEM((per_w * SEQ,), jnp.int32),
            pltpu.VMEM((CH * SEQ, EMBD), jnp.float32),
            pltpu.VMEM((CH * SEQ, EMBD), jnp.float32),
            pltpu.VMEM((CH, EMBD), jnp.float32),
            pltpu.VMEM((CH, EMBD), jnp.float32),
            pltpu.SemaphoreType.DMA,
            pltpu.SemaphoreType.DMA,
            pltpu.SemaphoreType.DMA,
            pltpu.SemaphoreType.DMA,
        ],
    )
    def pool_kernel(emb_hbm, idx_hbm, out_hbm, idx_v, rows0, rows1,
                    acc0, acc1, g0, g1, o0, o1):
        wid = lax.axis_index("s") * NCORES + lax.axis_index("c")
        base = wid * per_w
        pltpu.sync_copy(idx_hbm.at[pl.ds(base * SEQ, per_w * SEQ)], idx_v)

        def start_gather(c, rows, sem):
            pltpu.async_copy(
                emb_hbm.at[idx_v.at[pl.ds(c * (CH * SEQ), CH * SEQ)]],
                rows, sem)

        def wait_gather(rows, sem):
            pltpu.make_async_copy(
                emb_hbm.at[idx_v.at[pl.ds(0, CH * SEQ)]], rows, sem).wait()

        def start_out(c, acc, sem):
            pltpu.async_copy(acc, out_hbm.at[pl.ds(base + c * CH, CH)], sem)

        def wait_out(acc, sem):
            pltpu.make_async_copy(acc, out_hbm.at[pl.ds(base, CH)], sem).wait()

        def accumulate(rows, acc):
            for s in range(CH):
                r = s * SEQ
                accs = tuple(
                    rows[r, pl.ds(d * LANES, LANES)]
                    + rows[r + 1, pl.ds(d * LANES, LANES)]
                    for d in range(D)
                )

                def jbody(j, accs, r=r):
                    r0 = r + 2 * j
                    return tuple(
                        accs[d]
                        + rows[r0, pl.ds(d * LANES, LANES)]
                        + rows[r0 + 1, pl.ds(d * LANES, LANES)]
                        for d in range(D)
                    )

                accs = lax.fori_loop(1, SEQ // 2, jbody, accs)
                for d in range(D):
                    acc[s, pl.ds(d * LANES, LANES)] = accs[d]

        start_gather(0, rows0, g0)

        def pair(k, carry):
            c0 = 2 * k
            wait_gather(rows0, g0)
            start_gather(c0 + 1, rows1, g1)

            @pl.when(k > 0)
            def _():
                wait_out(acc0, o0)

            accumulate(rows0, acc0)
            start_out(c0, acc0, o0)
            wait_gather(rows1, g1)

            @pl.when(k < PAIRS - 1)
            def _():
                start_gather(c0 + 2, rows0, g0)

            @pl.when(k > 0)
            def _():
                wait_out(acc1, o1)

            accumulate(rows1, acc1)
            start_out(c0 + 1, acc1, o1)
            return carry

        lax.fori_loop(0, PAIRS, pair, 0)
        wait_out(acc0, o0)
        wait_out(acc1, o1)

    return pool_kernel(emb, idx_flat)


def _tc_mlp_cos(pooled, W1, b1, W2, b2, W3, b3, batch):
    TB = 512
    grid = batch // TB

    def body(ea_ref, eb_ref, w1, bb1, w2, bb2, w3, bb3, out_ref):
        def mlp(x):
            h = jnp.tanh(
                lax.dot_general(x, w1[...], (((1,), (1,)), ((), ())),
                                preferred_element_type=jnp.float32) + bb1[...])
            h = jnp.tanh(
                lax.dot_general(h, w2[...], (((1,), (1,)), ((), ())),
                                preferred_element_type=jnp.float32) + bb2[...])
            h = jnp.tanh(
                lax.dot_general(h, w3[...], (((1,), (1,)), ((), ())),
                                preferred_element_type=jnp.float32) + bb3[...])
            return h

        fa = mlp(ea_ref[...])
        fb = mlp(eb_ref[...])
        eps = 1e-8
        na = jnp.maximum(jnp.sqrt(jnp.sum(fa * fa, axis=1, keepdims=True)), eps)
        nb = jnp.maximum(jnp.sqrt(jnp.sum(fb * fb, axis=1, keepdims=True)), eps)
        dot = jnp.sum(fa * fb, axis=1, keepdims=True)
        out_ref[...] = dot / (na * nb)

    out = pl.pallas_call(
        body,
        grid=(grid,),
        in_specs=[
            pl.BlockSpec((TB, EMBD), lambda i: (i, 0)),
            pl.BlockSpec((TB, EMBD), lambda i: (i + grid, 0)),
            pl.BlockSpec(W1.shape, lambda i: (0, 0)),
            pl.BlockSpec((1, 256), lambda i: (0, 0)),
            pl.BlockSpec(W2.shape, lambda i: (0, 0)),
            pl.BlockSpec((1, 128), lambda i: (0, 0)),
            pl.BlockSpec(W3.shape, lambda i: (0, 0)),
            pl.BlockSpec((1, 64), lambda i: (0, 0)),
        ],
        out_specs=pl.BlockSpec((TB, 1), lambda i: (i, 0)),
        out_shape=jax.ShapeDtypeStruct((batch, 1), jnp.float32),
    )(pooled, pooled, W1, b1.reshape(1, -1), W2, b2.reshape(1, -1),
      W3, b3.reshape(1, -1))
    return out.reshape(-1)


def kernel(a, b, emb, W1, b1, W2, b2, W3, b3):
    batch = a.shape[0]
    idx = jnp.concatenate([a, b], axis=0).astype(jnp.int32).reshape(-1)
    pooled = _sc_pool(emb, idx, 2 * batch)
    return _tc_mlp_cos(pooled, W1, b1, W2, b2, W3, b3, batch)
